# baseline (speedup 1.0000x reference)
import functools

import jax
import jax.numpy as jnp
from jax import lax
from jax.experimental import pallas as pl
from jax.experimental.pallas import tpu as pltpu
from jax.experimental.pallas import tpu_sc as plsc

_N_PROMPTS = 32
_BATCH = 4
_SEQ = 2048
_HIDDEN = 1024

_info = plsc.get_sparse_core_info()
_NC = _info.num_cores
_NS = _info.num_subcores
_NW = _NC * _NS
_ROWS_PER_W = (_BATCH * _SEQ) // _NW
_CHUNK = 8
_NCHUNK = _ROWS_PER_W // _CHUNK
_NBUF = 12
_W_PER_B = _NW // _BATCH
_PROMPT_WORKERS = 16
_PROMPT_PER_W = (_BATCH * _N_PROMPTS) // _PROMPT_WORKERS

_mesh = plsc.VectorSubcoreMesh(core_axis_name="c", subcore_axis_name="s")


@functools.partial(
    pl.kernel,
    mesh=_mesh,
    out_type=jax.ShapeDtypeStruct((_BATCH, _N_PROMPTS + _SEQ, _HIDDEN), jnp.float32),
    scratch_types=[
        pltpu.VMEM((_ROWS_PER_W,), jnp.int32),
        pltpu.VMEM((_NBUF, _CHUNK, _HIDDEN), jnp.float32),
        pltpu.VMEM((_PROMPT_PER_W, _HIDDEN), jnp.float32),
    ] + [pltpu.SemaphoreType.DMA] * (2 * _NBUF + 1),
)
def _emb_lookup(
    ids_hbm, table_hbm, prompts_hbm, out_hbm,
    idx_v, rows_v, prompt_v,
    *sems,
):
    gsems = sems[:_NBUF]
    ssems = sems[_NBUF:2 * _NBUF]
    psem = sems[2 * _NBUF]
    wid = lax.axis_index("s") * _NC + lax.axis_index("c")
    b = wid // _W_PER_B
    sb = (wid % _W_PER_B) * _ROWS_PER_W

    is_prompt_worker = wid < _PROMPT_WORKERS
    pb = wid // (_PROMPT_WORKERS // _BATCH)
    p0 = (wid % (_PROMPT_WORKERS // _BATCH)) * _PROMPT_PER_W
    prompt_in = pltpu.make_async_copy(
        prompts_hbm.at[pl.ds(p0, _PROMPT_PER_W)], prompt_v, psem
    )

    @pl.when(is_prompt_worker)
    def _start_prompt_in():
        prompt_in.start()

    pltpu.sync_copy(ids_hbm.at[pl.ds(wid * _ROWS_PER_W, _ROWS_PER_W)], idx_v)

    def gather_start(c):
        return pltpu.make_async_copy(
            table_hbm.at[idx_v.at[pl.ds(c * _CHUNK, _CHUNK)]],
            rows_v.at[c % _NBUF],
            gsems[c % _NBUF],
        )

    def store_start(c):
        return pltpu.make_async_copy(
            rows_v.at[c % _NBUF],
            out_hbm.at[b, pl.ds(_N_PROMPTS + sb + c * _CHUNK, _CHUNK)],
            ssems[c % _NBUF],
        )

    gathers = [None] * _NCHUNK
    stores = [None] * _NCHUNK
    for c in range(min(_NBUF - 1, _NCHUNK)):
        gathers[c] = gather_start(c)
        gathers[c].start()

    for c in range(_NCHUNK):
        gathers[c].wait()
        stores[c] = store_start(c)
        stores[c].start()
        nxt = c + _NBUF - 1
        if nxt < _NCHUNK:
            prev = nxt - _NBUF
            if prev >= 0:
                stores[prev].wait()
            gathers[nxt] = gather_start(nxt)
            gathers[nxt].start()

    prompt_out = pltpu.make_async_copy(
        prompt_v, out_hbm.at[pb, pl.ds(p0, _PROMPT_PER_W)], psem
    )

    @pl.when(is_prompt_worker)
    def _prompt_out():
        prompt_in.wait()
        prompt_out.start()

    waited = set()
    for c in range(_NCHUNK):
        nxt = c + _NBUF - 1
        if nxt < _NCHUNK and nxt - _NBUF >= 0:
            waited.add(nxt - _NBUF)
    for c in range(_NCHUNK):
        if c not in waited:
            stores[c].wait()

    @pl.when(is_prompt_worker)
    def _prompt_drain():
        prompt_out.wait()


def kernel(prepadded_input_ids, emb_table, soft_prompts):
    ids = prepadded_input_ids[:, _N_PROMPTS:].reshape(-1)
    return _emb_lookup(ids, emb_table, soft_prompts)


# speedup vs baseline: 1.0248x; 1.0248x over previous
import functools

import jax
import jax.numpy as jnp
from jax import lax
from jax.experimental import pallas as pl
from jax.experimental.pallas import tpu as pltpu
from jax.experimental.pallas import tpu_sc as plsc

_N_PROMPTS = 32
_BATCH = 4
_SEQ = 2048
_HIDDEN = 1024

_info = plsc.get_sparse_core_info()
_NC = _info.num_cores
_NS = _info.num_subcores
_NW = _NC * _NS
_ROWS_PER_W = (_BATCH * _SEQ) // _NW
_CHUNK = 16
_NCHUNK = _ROWS_PER_W // _CHUNK
_NBUF = 7
_W_PER_B = _NW // _BATCH
_PROMPT_WORKERS = 16
_PROMPT_PER_W = (_BATCH * _N_PROMPTS) // _PROMPT_WORKERS

_mesh = plsc.VectorSubcoreMesh(core_axis_name="c", subcore_axis_name="s")


@functools.partial(
    pl.kernel,
    mesh=_mesh,
    out_type=jax.ShapeDtypeStruct((_BATCH, _N_PROMPTS + _SEQ, _HIDDEN), jnp.float32),
    scratch_types=[
        pltpu.VMEM((_ROWS_PER_W,), jnp.int32),
        pltpu.VMEM((_NBUF, _CHUNK, _HIDDEN), jnp.float32),
        pltpu.VMEM((_PROMPT_PER_W, _HIDDEN), jnp.float32),
    ] + [pltpu.SemaphoreType.DMA] * (2 * _NBUF + 1),
)
def _emb_lookup(
    ids_hbm, table_hbm, prompts_hbm, out_hbm,
    idx_v, rows_v, prompt_v,
    *sems,
):
    gsems = sems[:_NBUF]
    ssems = sems[_NBUF:2 * _NBUF]
    psem = sems[2 * _NBUF]
    wid = lax.axis_index("s") * _NC + lax.axis_index("c")
    b = wid // _W_PER_B
    sb = (wid % _W_PER_B) * _ROWS_PER_W

    is_prompt_worker = wid < _PROMPT_WORKERS
    pb = wid // (_PROMPT_WORKERS // _BATCH)
    p0 = (wid % (_PROMPT_WORKERS // _BATCH)) * _PROMPT_PER_W
    prompt_in = pltpu.make_async_copy(
        prompts_hbm.at[pl.ds(p0, _PROMPT_PER_W)], prompt_v, psem
    )

    @pl.when(is_prompt_worker)
    def _start_prompt_in():
        prompt_in.start()

    pltpu.sync_copy(ids_hbm.at[pl.ds(wid * _ROWS_PER_W, _ROWS_PER_W)], idx_v)

    def gather_start(c):
        return pltpu.make_async_copy(
            table_hbm.at[idx_v.at[pl.ds(c * _CHUNK, _CHUNK)]],
            rows_v.at[c % _NBUF],
            gsems[c % _NBUF],
        )

    def store_start(c):
        return pltpu.make_async_copy(
            rows_v.at[c % _NBUF],
            out_hbm.at[b, pl.ds(_N_PROMPTS + sb + c * _CHUNK, _CHUNK)],
            ssems[c % _NBUF],
        )

    gathers = [None] * _NCHUNK
    stores = [None] * _NCHUNK
    for c in range(min(_NBUF - 1, _NCHUNK)):
        gathers[c] = gather_start(c)
        gathers[c].start()

    for c in range(_NCHUNK):
        gathers[c].wait()
        stores[c] = store_start(c)
        stores[c].start()
        nxt = c + _NBUF - 1
        if nxt < _NCHUNK:
            prev = nxt - _NBUF
            if prev >= 0:
                stores[prev].wait()
            gathers[nxt] = gather_start(nxt)
            gathers[nxt].start()

    prompt_out = pltpu.make_async_copy(
        prompt_v, out_hbm.at[pb, pl.ds(p0, _PROMPT_PER_W)], psem
    )

    @pl.when(is_prompt_worker)
    def _prompt_out():
        prompt_in.wait()
        prompt_out.start()

    waited = set()
    for c in range(_NCHUNK):
        nxt = c + _NBUF - 1
        if nxt < _NCHUNK and nxt - _NBUF >= 0:
            waited.add(nxt - _NBUF)
    for c in range(_NCHUNK):
        if c not in waited:
            stores[c].wait()

    @pl.when(is_prompt_worker)
    def _prompt_drain():
        prompt_out.wait()


def kernel(prepadded_input_ids, emb_table, soft_prompts):
    ids = prepadded_input_ids[:, _N_PROMPTS:].reshape(-1)
    return _emb_lookup(ids, emb_table, soft_prompts)


# speedup vs baseline: 1.0299x; 1.0049x over previous
import functools

import jax
import jax.numpy as jnp
from jax import lax
from jax.experimental import pallas as pl
from jax.experimental.pallas import tpu as pltpu
from jax.experimental.pallas import tpu_sc as plsc

_N_PROMPTS = 32
_BATCH = 4
_SEQ = 2048
_HIDDEN = 1024

_info = plsc.get_sparse_core_info()
_NC = _info.num_cores
_NS = _info.num_subcores
_NW = _NC * _NS
_ROWS_PER_W = (_BATCH * _SEQ) // _NW
_CHUNK = 16
_NCHUNK = _ROWS_PER_W // _CHUNK
_NBUF = 6
_W_PER_B = _NW // _BATCH
_PROMPT_WORKERS = 16
_PROMPT_PER_W = (_BATCH * _N_PROMPTS) // _PROMPT_WORKERS

_mesh = plsc.VectorSubcoreMesh(core_axis_name="c", subcore_axis_name="s")


@functools.partial(
    pl.kernel,
    mesh=_mesh,
    out_type=jax.ShapeDtypeStruct((_BATCH, _N_PROMPTS + _SEQ, _HIDDEN), jnp.float32),
    scratch_types=[
        pltpu.VMEM((_ROWS_PER_W,), jnp.int32),
        pltpu.VMEM((_NBUF, _CHUNK, _HIDDEN), jnp.float32),
        pltpu.VMEM((_PROMPT_PER_W, _HIDDEN), jnp.float32),
    ] + [pltpu.SemaphoreType.DMA] * (2 * _NBUF + 1),
)
def _emb_lookup(
    ids_hbm, table_hbm, prompts_hbm, out_hbm,
    idx_v, rows_v, prompt_v,
    *sems,
):
    gsems = sems[:_NBUF]
    ssems = sems[_NBUF:2 * _NBUF]
    psem = sems[2 * _NBUF]
    wid = lax.axis_index("s") * _NC + lax.axis_index("c")
    b = wid // _W_PER_B
    sb = (wid % _W_PER_B) * _ROWS_PER_W

    is_prompt_worker = wid < _PROMPT_WORKERS
    pb = wid // (_PROMPT_WORKERS // _BATCH)
    p0 = (wid % (_PROMPT_WORKERS // _BATCH)) * _PROMPT_PER_W
    prompt_in = pltpu.make_async_copy(
        prompts_hbm.at[pl.ds(p0, _PROMPT_PER_W)], prompt_v, psem
    )

    @pl.when(is_prompt_worker)
    def _start_prompt_in():
        prompt_in.start()

    pltpu.sync_copy(ids_hbm.at[pl.ds(wid * _ROWS_PER_W, _ROWS_PER_W)], idx_v)

    def gather_start(c):
        return pltpu.make_async_copy(
            table_hbm.at[idx_v.at[pl.ds(c * _CHUNK, _CHUNK)]],
            rows_v.at[c % _NBUF],
            gsems[c % _NBUF],
        )

    def store_start(c):
        return pltpu.make_async_copy(
            rows_v.at[c % _NBUF],
            out_hbm.at[b, pl.ds(_N_PROMPTS + sb + c * _CHUNK, _CHUNK)],
            ssems[c % _NBUF],
        )

    gathers = [None] * _NCHUNK
    stores = [None] * _NCHUNK
    for c in range(min(_NBUF - 1, _NCHUNK)):
        gathers[c] = gather_start(c)
        gathers[c].start()

    for c in range(_NCHUNK):
        gathers[c].wait()
        stores[c] = store_start(c)
        stores[c].start()
        nxt = c + _NBUF - 1
        if nxt < _NCHUNK:
            prev = nxt - _NBUF
            if prev >= 0:
                stores[prev].wait()
            gathers[nxt] = gather_start(nxt)
            gathers[nxt].start()

    prompt_out = pltpu.make_async_copy(
        prompt_v, out_hbm.at[pb, pl.ds(p0, _PROMPT_PER_W)], psem
    )

    @pl.when(is_prompt_worker)
    def _prompt_out():
        prompt_in.wait()
        prompt_out.start()

    waited = set()
    for c in range(_NCHUNK):
        nxt = c + _NBUF - 1
        if nxt < _NCHUNK and nxt - _NBUF >= 0:
            waited.add(nxt - _NBUF)
    for c in range(_NCHUNK):
        if c not in waited:
            stores[c].wait()

    @pl.when(is_prompt_worker)
    def _prompt_drain():
        prompt_out.wait()


def kernel(prepadded_input_ids, emb_table, soft_prompts):
    ids = prepadded_input_ids[:, _N_PROMPTS:].reshape(-1)
    return _emb_lookup(ids, emb_table, soft_prompts)
